# baseline (speedup 1.0000x reference)
import functools
import jax
import jax.numpy as jnp
from jax import lax
from jax.experimental import pallas as pl
from jax.experimental.pallas import tpu as pltpu
from jax.experimental.pallas import tpu_sc as plsc

B, N, D = 128, 8192, 64
NW = 32
RPW = B // NW
C = 256
CW = C * D
NCH = N // C
GRP = C // 16


def _sc_body(x_hbm, out_hbm, buf0, buf1, rowbuf, sem0, sem1):
    cid = lax.axis_index("c")
    sid = lax.axis_index("s")
    wid = sid * 2 + cid
    iota = lax.iota(jnp.int32, 16)

    def compute_chunk(buf, cap0, carry):
        def group(g, carry):
            bv, bi = carry
            cap = g * 16 + iota
            z = jnp.zeros((16,), jnp.float32)
            accs = [z, z, z, z]
            for d in range(D):
                val = plsc.load_gather(buf, [cap, jnp.full((16,), d, jnp.int32)])
                accs[d % 4] = accs[d % 4] + val * val
            s = (accs[0] + accs[1]) + (accs[2] + accs[3])
            gcap = cap0 + cap
            better = s > bv
            return jnp.where(better, s, bv), jnp.where(better, gcap, bi)

        return lax.fori_loop(0, GRP, group, carry)

    for j in range(RPW):
        b = wid * RPW + j
        pltpu.async_copy(x_hbm.at[b, pl.ds(0, C)], buf0, sem0)
        pltpu.async_copy(x_hbm.at[b, pl.ds(C, C)], buf1, sem1)

        def chunk_pair(i, carry, b=b):
            pltpu.make_async_copy(x_hbm.at[b, pl.ds(0, C)], buf0, sem0).wait()
            carry = compute_chunk(buf0, 2 * i * C, carry)

            @pl.when(i < NCH // 2 - 1)
            def _():
                pltpu.async_copy(x_hbm.at[b, pl.ds((2 * i + 2) * C, C)], buf0, sem0)

            pltpu.make_async_copy(x_hbm.at[b, pl.ds(0, C)], buf1, sem1).wait()
            carry = compute_chunk(buf1, (2 * i + 1) * C, carry)

            @pl.when(i < NCH // 2 - 1)
            def _():
                pltpu.async_copy(x_hbm.at[b, pl.ds((2 * i + 3) * C, C)], buf1, sem1)

            return carry

        init = (jnp.full((16,), -1.0, jnp.float32), jnp.zeros((16,), jnp.int32))
        bv, bi = lax.fori_loop(0, NCH // 2, chunk_pair, init)

        m = jnp.max(bv)
        cand = jnp.where(bv == m, bi, jnp.int32(1 << 30))
        win = jnp.min(cand)
        pltpu.sync_copy(x_hbm.at[b, pl.ds(win, 1)], rowbuf.at[pl.ds(j, 1)])

    pltpu.sync_copy(rowbuf, out_hbm.at[pl.ds(wid * RPW, RPW)])


_sc_kernel = functools.partial(
    pl.kernel,
    mesh=plsc.VectorSubcoreMesh(core_axis_name="c", subcore_axis_name="s"),
    compiler_params=pltpu.CompilerParams(needs_layout_passes=False),
    out_type=jax.ShapeDtypeStruct((B, D), jnp.float32),
    scratch_types=[
        pltpu.VMEM((C, D), jnp.float32),
        pltpu.VMEM((C, D), jnp.float32),
        pltpu.VMEM((RPW, D), jnp.float32),
        pltpu.SemaphoreType.DMA,
        pltpu.SemaphoreType.DMA,
    ],
)(_sc_body)


def kernel(input):
    return _sc_kernel(input)


# speedup vs baseline: 1.7755x; 1.7755x over previous
import functools
import jax
import jax.numpy as jnp
from jax import lax
from jax.experimental import pallas as pl
from jax.experimental.pallas import tpu as pltpu
from jax.experimental.pallas import tpu_sc as plsc

B, N, D = 128, 8192, 64
NW = 32
RPW = B // NW
C = 256
CW = C * D
NCH = N // C
GRP = C // 16


def _sc_body(x_hbm, out_hbm, buf0, buf1, rowbuf, sem0, sem1):
    cid = lax.axis_index("c")
    sid = lax.axis_index("s")
    wid = sid * 2 + cid
    iota = lax.iota(jnp.int32, 16)

    def compute_chunk(buf, cap0, carry):
        def group(g, carry):
            bv, bi = carry
            cap = g * 16 + iota
            z = jnp.zeros((16,), jnp.float32)
            accs = [z, z, z, z]
            for d in range(D):
                dvec = (iota + d) & (D - 1)
                val = plsc.load_gather(buf, [cap, dvec])
                accs[d % 4] = accs[d % 4] + val * val
            s = (accs[0] + accs[1]) + (accs[2] + accs[3])
            gcap = cap0 + cap
            better = s > bv
            return jnp.where(better, s, bv), jnp.where(better, gcap, bi)

        return lax.fori_loop(0, GRP, group, carry)

    for j in range(RPW):
        b = wid * RPW + j
        pltpu.async_copy(x_hbm.at[b, pl.ds(0, C)], buf0, sem0)
        pltpu.async_copy(x_hbm.at[b, pl.ds(C, C)], buf1, sem1)

        def chunk_pair(i, carry, b=b):
            pltpu.make_async_copy(x_hbm.at[b, pl.ds(0, C)], buf0, sem0).wait()
            carry = compute_chunk(buf0, 2 * i * C, carry)

            @pl.when(i < NCH // 2 - 1)
            def _():
                pltpu.async_copy(x_hbm.at[b, pl.ds((2 * i + 2) * C, C)], buf0, sem0)

            pltpu.make_async_copy(x_hbm.at[b, pl.ds(0, C)], buf1, sem1).wait()
            carry = compute_chunk(buf1, (2 * i + 1) * C, carry)

            @pl.when(i < NCH // 2 - 1)
            def _():
                pltpu.async_copy(x_hbm.at[b, pl.ds((2 * i + 3) * C, C)], buf1, sem1)

            return carry

        init = (jnp.full((16,), -1.0, jnp.float32), jnp.zeros((16,), jnp.int32))
        bv, bi = lax.fori_loop(0, NCH // 2, chunk_pair, init)

        m = jnp.max(bv)
        cand = jnp.where(bv == m, bi, jnp.int32(1 << 30))
        win = jnp.min(cand)
        pltpu.sync_copy(x_hbm.at[b, pl.ds(win, 1)], rowbuf.at[pl.ds(j, 1)])

    pltpu.sync_copy(rowbuf, out_hbm.at[pl.ds(wid * RPW, RPW)])


_sc_kernel = functools.partial(
    pl.kernel,
    mesh=plsc.VectorSubcoreMesh(core_axis_name="c", subcore_axis_name="s"),
    compiler_params=pltpu.CompilerParams(needs_layout_passes=False),
    out_type=jax.ShapeDtypeStruct((B, D), jnp.float32),
    scratch_types=[
        pltpu.VMEM((C, D), jnp.float32),
        pltpu.VMEM((C, D), jnp.float32),
        pltpu.VMEM((RPW, D), jnp.float32),
        pltpu.SemaphoreType.DMA,
        pltpu.SemaphoreType.DMA,
    ],
)(_sc_body)


def kernel(input):
    return _sc_kernel(input)
